# Initial kernel scaffold; baseline (speedup 1.0000x reference)
import jax
import jax.numpy as jnp
from jax.experimental import pallas as pl


def kernel(x):
    raise NotImplementedError("write your pallas kernel here")


# speedup vs baseline: 56.3442x; 56.3442x over previous
import jax
import jax.numpy as jnp
from jax import lax
from jax.experimental import pallas as pl

B, L = 1024, 50


def _tc_body(x_ref, o_ref):
    x = x_ref[...]
    pos = lax.broadcasted_iota(jnp.int32, (B, L), 1)

    acc = jnp.zeros((B, L), jnp.float32)
    for j in range(L):
        eq = (x == x[:, j : j + 1]) & (pos >= j)
        acc = acc + eq.astype(jnp.float32)
    o_ref[...] = acc


def kernel(x):
    out = pl.pallas_call(
        _tc_body,
        out_shape=jax.ShapeDtypeStruct((B, L), jnp.float32),
    )(x.astype(jnp.int32))
    return out[..., None]
